# baseline (speedup 1.0000x reference)
import functools

import jax
import jax.numpy as jnp
from jax import lax
from jax.experimental import pallas as pl
from jax.experimental.pallas import tpu as pltpu
from jax.experimental.pallas import tpu_sc as plsc

N = 10000
E = 320000
D = 128
BN_EPS = 1e-5

NC = 2
NS = 16
NW = NC * NS
EDGES_PER_TILE = E // NW
CHUNK = 64
NCHUNK = 160
EDGES_PER_TILE_PAD = NCHUNK * CHUNK
GROUP = 5
NGROUP = NCHUNK // GROUP
NBUF = 4
ACC_ROWS = N + 8
ROW_STRIDE = 624
ROW_WIN = 640


def _sc_segment_sum(features, src3, dst3, zeros):
    mesh = plsc.VectorSubcoreMesh(core_axis_name="c", subcore_axis_name="s")

    @functools.partial(
        pl.kernel,
        out_type=jax.ShapeDtypeStruct((NC, N, D), jnp.float32),
        mesh=mesh,
        scratch_types=[
            pltpu.VMEM((2, GROUP, CHUNK), jnp.int32),
            pltpu.VMEM((2, GROUP, CHUNK), jnp.int32),
            pltpu.VMEM((NBUF, CHUNK, D), jnp.float32),
            pltpu.VMEM_SHARED((ACC_ROWS, D), jnp.float32),
            pltpu.SemaphoreType.DMA((NBUF,)),
            pltpu.SemaphoreType.DMA((2,)),
        ],
    )
    def k(features_hbm, src_hbm, dst_hbm, zeros_hbm, out_hbm,
          src_idx, dst_idx, rows, acc, gsem, isem):
        cid = lax.axis_index("c")
        sid = lax.axis_index("s")
        wid = cid * NS + sid

        base = sid * ROW_STRIDE
        pltpu.sync_copy(zeros_hbm.at[pl.ds(base, ROW_WIN)],
                        acc.at[pl.ds(base, ROW_WIN)])
        pltpu.sync_copy(src_hbm.at[wid, 0], src_idx.at[0])
        pltpu.sync_copy(dst_hbm.at[wid, 0], dst_idx.at[0])
        plsc.subcore_barrier()

        for w in range(NBUF - 1):
            pltpu.async_copy(features_hbm.at[src_idx.at[0, w]], rows.at[w],
                             gsem.at[w])

        def body(j, carry):
            g = j // GROUP
            r = j % GROUP
            pg = g % 2
            pj = j % NBUF
            nxt = j + NBUF - 1

            @pl.when(jnp.logical_and(r == 0, g < NGROUP - 1))
            def _stage_next_group():
                npg = (g + 1) % 2
                pltpu.async_copy(src_hbm.at[wid, g + 1], src_idx.at[npg],
                                 isem.at[npg])
                pltpu.async_copy(dst_hbm.at[wid, g + 1], dst_idx.at[npg],
                                 isem.at[npg])

            @pl.when(jnp.logical_and(r == GROUP - NBUF + 1, g < NGROUP - 1))
            def _wait_next_group():
                npg = (g + 1) % 2
                pltpu.make_async_copy(src_hbm.at[wid, g + 1], src_idx.at[npg],
                                      isem.at[npg]).wait()
                pltpu.make_async_copy(dst_hbm.at[wid, g + 1], dst_idx.at[npg],
                                      isem.at[npg]).wait()

            @pl.when(nxt < NCHUNK)
            def _fire_next_gather():
                pltpu.async_copy(
                    features_hbm.at[src_idx.at[(nxt // GROUP) % 2,
                                               nxt % GROUP]],
                    rows.at[nxt % NBUF], gsem.at[nxt % NBUF])

            pltpu.make_async_copy(features_hbm.at[src_idx.at[pg, r]],
                                  rows.at[pj], gsem.at[pj]).wait()
            pltpu.sync_copy(rows.at[pj], acc.at[dst_idx.at[pg, r]], add=True)
            return carry

        lax.fori_loop(0, NCHUNK, body, 0)

        plsc.subcore_barrier()
        pltpu.sync_copy(acc.at[pl.ds(base, ROW_WIN)],
                        out_hbm.at[cid, pl.ds(base, ROW_WIN)])

    return k(features, src3, dst3, zeros)


def _tc_body(f_ref, p_ref, w_ref, b_ref, g_ref, be_ref, o_ref):
    f = f_ref[...]
    h = f + p_ref[0] + p_ref[1]
    y = jnp.dot(h, w_ref[...], preferred_element_type=jnp.float32) + b_ref[...]
    mean = jnp.mean(y, axis=0, keepdims=True)
    c = y - mean
    var = jnp.mean(c * c, axis=0, keepdims=True)
    yn = c * lax.rsqrt(var + BN_EPS) * g_ref[...] + be_ref[...]
    o_ref[...] = f + jnp.maximum(yn, 0.0)


def kernel(features, edge_index, norm, W, b, gamma, beta):
    del norm
    pad = EDGES_PER_TILE_PAD - EDGES_PER_TILE
    src3 = jnp.pad(edge_index[0].reshape(NW, EDGES_PER_TILE), ((0, 0), (0, pad)),
                   constant_values=0).reshape(NW, NGROUP, GROUP, CHUNK)
    dst3 = jnp.pad(edge_index[1].reshape(NW, EDGES_PER_TILE), ((0, 0), (0, pad)),
                   constant_values=N).reshape(NW, NGROUP, GROUP, CHUNK)
    zeros = jnp.zeros((N, D), jnp.float32)
    partials = _sc_segment_sum(features, src3, dst3, zeros)
    return pl.pallas_call(
        _tc_body,
        out_shape=jax.ShapeDtypeStruct((N, D), jnp.float32),
    )(features, partials, W, b.reshape(1, D), gamma.reshape(1, D),
      beta.reshape(1, D))


# speedup vs baseline: 2.9765x; 2.9765x over previous
import functools

import jax
import jax.numpy as jnp
from jax import lax
from jax.experimental import pallas as pl
from jax.experimental.pallas import tpu as pltpu
from jax.experimental.pallas import tpu_sc as plsc

N = 10000
E = 320000
D = 128
BN_EPS = 1e-5

NC = 2
NS = 16
NW = NC * NS
EDGES_PER_TILE = E // NW
CHUNK = 80
NCHUNK = EDGES_PER_TILE // CHUNK
GROUP = 5
NGROUP = NCHUNK // GROUP
NBUF = 3
ACC_ROWS = N
ROW_STRIDE = 624
ROW_WIN = 640


def _sc_segment_sum(features, src3, dst3, zeros):
    mesh = plsc.VectorSubcoreMesh(core_axis_name="c", subcore_axis_name="s")

    @functools.partial(
        pl.kernel,
        out_type=jax.ShapeDtypeStruct((NC, N, D), jnp.float32),
        mesh=mesh,
        scratch_types=[
            pltpu.VMEM((2, GROUP, CHUNK), jnp.int32),
            pltpu.VMEM((2, GROUP, CHUNK), jnp.int32),
            pltpu.VMEM((NBUF, CHUNK, D), jnp.float32),
            pltpu.VMEM_SHARED((ACC_ROWS, D), jnp.float32),
            pltpu.SemaphoreType.DMA((NBUF,)),
            pltpu.SemaphoreType.DMA((NBUF,)),
            pltpu.SemaphoreType.DMA((2,)),
        ],
    )
    def k(features_hbm, src_hbm, dst_hbm, zeros_hbm, out_hbm,
          src_idx, dst_idx, rows, acc, gsem, ssem, isem):
        cid = lax.axis_index("c")
        sid = lax.axis_index("s")
        wid = cid * NS + sid

        base = sid * ROW_STRIDE
        pltpu.sync_copy(zeros_hbm.at[pl.ds(base, ROW_WIN)],
                        acc.at[pl.ds(base, ROW_WIN)])
        pltpu.sync_copy(src_hbm.at[wid, 0], src_idx.at[0])
        pltpu.sync_copy(dst_hbm.at[wid, 0], dst_idx.at[0])
        plsc.subcore_barrier()

        for w in range(NBUF - 1):
            pltpu.async_copy(features_hbm.at[src_idx.at[0, w]], rows.at[w],
                             gsem.at[w])

        def body(j, carry):
            g = j // GROUP
            r = j % GROUP
            pg = g % 2
            pj = j % NBUF
            nxt = j + NBUF - 1

            @pl.when(jnp.logical_and(r == 0, g < NGROUP - 1))
            def _stage_next_group():
                npg = (g + 1) % 2
                pltpu.async_copy(src_hbm.at[wid, g + 1], src_idx.at[npg],
                                 isem.at[npg])
                pltpu.async_copy(dst_hbm.at[wid, g + 1], dst_idx.at[npg],
                                 isem.at[npg])

            @pl.when(jnp.logical_and(r == GROUP - NBUF + 1, g < NGROUP - 1))
            def _wait_next_group():
                npg = (g + 1) % 2
                pltpu.make_async_copy(src_hbm.at[wid, g + 1], src_idx.at[npg],
                                      isem.at[npg]).wait()
                pltpu.make_async_copy(dst_hbm.at[wid, g + 1], dst_idx.at[npg],
                                      isem.at[npg]).wait()

            @pl.when(nxt < NCHUNK)
            def _fire_next_gather():
                @pl.when(j >= 1)
                def _reuse_wait():
                    pltpu.make_async_copy(rows.at[nxt % NBUF],
                                          acc.at[dst_idx.at[pg, r]],
                                          ssem.at[nxt % NBUF]).wait()
                pltpu.async_copy(
                    features_hbm.at[src_idx.at[(nxt // GROUP) % 2,
                                               nxt % GROUP]],
                    rows.at[nxt % NBUF], gsem.at[nxt % NBUF])

            pltpu.make_async_copy(features_hbm.at[src_idx.at[pg, r]],
                                  rows.at[pj], gsem.at[pj]).wait()
            pltpu.async_copy(rows.at[pj], acc.at[dst_idx.at[pg, r]],
                             ssem.at[pj], add=True)
            return carry

        lax.fori_loop(0, NCHUNK, body, 0)
        for c in range(NCHUNK - NBUF, NCHUNK):
            pltpu.make_async_copy(rows.at[c % NBUF],
                                  acc.at[dst_idx.at[(c // GROUP) % 2,
                                                    c % GROUP]],
                                  ssem.at[c % NBUF]).wait()

        plsc.subcore_barrier()
        pltpu.sync_copy(acc.at[pl.ds(base, ROW_WIN)],
                        out_hbm.at[cid, pl.ds(base, ROW_WIN)])

    return k(features, src3, dst3, zeros)


def _tc_body(f_ref, p_ref, w_ref, b_ref, g_ref, be_ref, o_ref):
    f = f_ref[...]
    h = f + p_ref[0] + p_ref[1]
    y = jnp.dot(h, w_ref[...], preferred_element_type=jnp.float32) + b_ref[...]
    mean = jnp.mean(y, axis=0, keepdims=True)
    c = y - mean
    var = jnp.mean(c * c, axis=0, keepdims=True)
    yn = c * lax.rsqrt(var + BN_EPS) * g_ref[...] + be_ref[...]
    o_ref[...] = f + jnp.maximum(yn, 0.0)


def kernel(features, edge_index, norm, W, b, gamma, beta):
    del norm
    src3 = edge_index[0].reshape(NW, NGROUP, GROUP, CHUNK)
    dst3 = edge_index[1].reshape(NW, NGROUP, GROUP, CHUNK)
    zeros = jnp.zeros((N, D), jnp.float32)
    partials = _sc_segment_sum(features, src3, dst3, zeros)
    return pl.pallas_call(
        _tc_body,
        out_shape=jax.ShapeDtypeStruct((N, D), jnp.float32),
    )(features, partials, W, b.reshape(1, D), gamma.reshape(1, D),
      beta.reshape(1, D))


# speedup vs baseline: 3.1881x; 1.0711x over previous
import functools

import jax
import jax.numpy as jnp
from jax import lax
from jax.experimental import pallas as pl
from jax.experimental.pallas import tpu as pltpu
from jax.experimental.pallas import tpu_sc as plsc

N = 10000
E = 320000
D = 128
BN_EPS = 1e-5

NC = 2
NS = 16
NW = NC * NS
EDGES_PER_TILE = E // NW
CHUNK = 80
NCHUNK = EDGES_PER_TILE // CHUNK
GROUP = 5
NGROUP = NCHUNK // GROUP
NBUF = 3
ACC_ROWS = N
ROW_STRIDE = 624
ROW_WIN = 640


ZROWS = 32


def _sc_segment_sum(features, src_flat, dst_flat):
    mesh = plsc.VectorSubcoreMesh(core_axis_name="c", subcore_axis_name="s")

    @functools.partial(
        pl.kernel,
        out_type=jax.ShapeDtypeStruct((NC, N, D), jnp.float32),
        mesh=mesh,
        scratch_types=[
            pltpu.VMEM((2, GROUP, CHUNK), jnp.int32),
            pltpu.VMEM((2, GROUP, CHUNK), jnp.int32),
            pltpu.VMEM((NBUF, CHUNK, D), jnp.float32),
            pltpu.VMEM((ZROWS, D), jnp.float32),
            pltpu.VMEM_SHARED((ACC_ROWS, D), jnp.float32),
            pltpu.SemaphoreType.DMA((NBUF,)),
            pltpu.SemaphoreType.DMA((NBUF,)),
            pltpu.SemaphoreType.DMA((2,)),
        ],
    )
    def k(features_hbm, src_hbm, dst_hbm, out_hbm,
          src_idx, dst_idx, rows, zbuf, acc, gsem, ssem, isem):
        cid = lax.axis_index("c")
        sid = lax.axis_index("s")
        wid = cid * NS + sid
        ebase = wid * EDGES_PER_TILE

        def _stage_group(g, sem):
            for rr in range(GROUP):
                off = pl.multiple_of(ebase + g * (GROUP * CHUNK) + rr * CHUNK,
                                     16)
                pltpu.async_copy(src_hbm.at[pl.ds(off, CHUNK)],
                                 src_idx.at[g % 2, rr], sem)
                pltpu.async_copy(dst_hbm.at[pl.ds(off, CHUNK)],
                                 dst_idx.at[g % 2, rr], sem)

        def _wait_group(g, sem):
            for rr in range(GROUP):
                off = pl.multiple_of(ebase + g * (GROUP * CHUNK) + rr * CHUNK,
                                     16)
                pltpu.make_async_copy(src_hbm.at[pl.ds(off, CHUNK)],
                                      src_idx.at[g % 2, rr], sem).wait()
                pltpu.make_async_copy(dst_hbm.at[pl.ds(off, CHUNK)],
                                      dst_idx.at[g % 2, rr], sem).wait()

        _stage_group(0, isem.at[0])

        def zstore(i, carry):
            zbuf[i // (D // 16), pl.ds((i % (D // 16)) * 16, 16)] = (
                jnp.zeros((16,), jnp.float32))
            return carry

        lax.fori_loop(0, ZROWS * (D // 16), zstore, 0)
        base = sid * ROW_STRIDE
        for q in range(ROW_WIN // ZROWS):
            pltpu.sync_copy(zbuf, acc.at[pl.ds(base + q * ZROWS, ZROWS)])

        _wait_group(0, isem.at[0])
        plsc.subcore_barrier()

        for w in range(NBUF - 1):
            pltpu.async_copy(features_hbm.at[src_idx.at[0, w]], rows.at[w],
                             gsem.at[w])

        def body(j, carry):
            g = j // GROUP
            r = j % GROUP
            pg = g % 2
            pj = j % NBUF
            nxt = j + NBUF - 1

            @pl.when(jnp.logical_and(r == 0, g < NGROUP - 1))
            def _stage_next_group():
                _stage_group(g + 1, isem.at[(g + 1) % 2])

            @pl.when(jnp.logical_and(r == GROUP - NBUF + 1, g < NGROUP - 1))
            def _wait_next_group():
                _wait_group(g + 1, isem.at[(g + 1) % 2])

            @pl.when(nxt < NCHUNK)
            def _fire_next_gather():
                @pl.when(j >= 1)
                def _reuse_wait():
                    pltpu.make_async_copy(rows.at[nxt % NBUF],
                                          acc.at[dst_idx.at[pg, r]],
                                          ssem.at[nxt % NBUF]).wait()
                pltpu.async_copy(
                    features_hbm.at[src_idx.at[(nxt // GROUP) % 2,
                                               nxt % GROUP]],
                    rows.at[nxt % NBUF], gsem.at[nxt % NBUF])

            pltpu.make_async_copy(features_hbm.at[src_idx.at[pg, r]],
                                  rows.at[pj], gsem.at[pj]).wait()
            pltpu.async_copy(rows.at[pj], acc.at[dst_idx.at[pg, r]],
                             ssem.at[pj], add=True)
            return carry

        lax.fori_loop(0, NCHUNK, body, 0)
        for c in range(NCHUNK - NBUF, NCHUNK):
            pltpu.make_async_copy(rows.at[c % NBUF],
                                  acc.at[dst_idx.at[(c // GROUP) % 2,
                                                    c % GROUP]],
                                  ssem.at[c % NBUF]).wait()

        plsc.subcore_barrier()
        pltpu.sync_copy(acc.at[pl.ds(base, ROW_WIN)],
                        out_hbm.at[cid, pl.ds(base, ROW_WIN)])

    return k(features, src_flat, dst_flat)


def _tc_body(f_ref, p_ref, w_ref, b_ref, g_ref, be_ref, o_ref):
    f = f_ref[...]
    h = f + (p_ref[0] + p_ref[1])
    y = jnp.dot(h, w_ref[...], preferred_element_type=jnp.float32) + b_ref[...]
    mean = jnp.mean(y, axis=0, keepdims=True)
    c = y - mean
    var = jnp.mean(c * c, axis=0, keepdims=True)
    yn = c * lax.rsqrt(var + BN_EPS) * g_ref[...] + be_ref[...]
    o_ref[...] = f + jnp.maximum(yn, 0.0)


def kernel(features, edge_index, norm, W, b, gamma, beta):
    del norm
    partials = _sc_segment_sum(features, edge_index[0], edge_index[1])
    return pl.pallas_call(
        _tc_body,
        out_shape=jax.ShapeDtypeStruct((N, D), jnp.float32),
    )(features, partials, W, b.reshape(1, D), gamma.reshape(1, D),
      beta.reshape(1, D))
